# baseline (speedup 1.0000x reference)
import functools
import math

import jax
import jax.numpy as jnp
from jax import lax
from jax.experimental import pallas as pl
from jax.experimental.pallas import tpu as pltpu
from jax.experimental.pallas import tpu_sc as plsc

CHUNK = 128
_info = plsc.get_sparse_core_info()
NC, NS = _info.num_cores, _info.num_subcores
NW = NC * NS


@functools.lru_cache(maxsize=None)
def _make_sc_lookup(num_chunks, vocab, d):
    scale = math.sqrt(d)
    mesh = plsc.VectorSubcoreMesh(core_axis_name="c", subcore_axis_name="s")

    @functools.partial(
        pl.kernel,
        mesh=mesh,
        out_type=jax.ShapeDtypeStruct((NW * num_chunks * CHUNK, d), jnp.float32),
        scratch_types=[
            pltpu.VMEM((num_chunks, CHUNK), jnp.int32),
            pltpu.VMEM((CHUNK, d), jnp.float32),
            pltpu.SemaphoreType.DMA,
        ],
        compiler_params=pltpu.CompilerParams(use_tc_tiling_on_sc=False),
    )
    def k(idx_hbm, table_hbm, out_hbm, idx_v, rows_v, sem):
        wid = lax.axis_index("s") * NC + lax.axis_index("c")
        base = wid * (num_chunks * CHUNK)
        pltpu.sync_copy(idx_hbm.at[wid], idx_v)

        def chunk_body(j, carry):
            pltpu.async_copy(table_hbm.at[idx_v.at[j]], rows_v, sem).wait()

            def scale_body(i, c):
                for t in range(d // 16):
                    sl = pl.ds(t * 16, 16)
                    rows_v[i, sl] = rows_v[i, sl] * scale
                return c

            lax.fori_loop(0, CHUNK, scale_body, 0, unroll=2)
            pltpu.sync_copy(rows_v, out_hbm.at[pl.ds(base + j * CHUNK, CHUNK)])
            return carry

        lax.fori_loop(0, num_chunks, chunk_body, 0)

    return k


def kernel(x, table):
    d = table.shape[1]
    b_total = x.size
    xf = x.reshape(-1).astype(jnp.int32)
    block = NW * CHUNK
    pad = (-b_total) % block
    if pad:
        xf = jnp.concatenate([xf, jnp.zeros((pad,), jnp.int32)])
    num_chunks = xf.size // block
    idx = xf.reshape(NW, num_chunks, CHUNK)
    out = _make_sc_lookup(num_chunks, table.shape[0], d)(idx, table)
    if pad:
        out = out[:b_total]
    return out.reshape(x.shape + (d,))


# speedup vs baseline: 1.1617x; 1.1617x over previous
import functools
import math

import jax
import jax.numpy as jnp
from jax import lax
from jax.experimental import pallas as pl
from jax.experimental.pallas import tpu as pltpu
from jax.experimental.pallas import tpu_sc as plsc

CHUNK = 128
RING = 4
_info = plsc.get_sparse_core_info()
NC, NS = _info.num_cores, _info.num_subcores
NW = NC * NS


@functools.lru_cache(maxsize=None)
def _make_sc_lookup(num_chunks, vocab, d):
    scale = math.sqrt(d)
    mesh = plsc.VectorSubcoreMesh(core_axis_name="c", subcore_axis_name="s")
    assert num_chunks % RING == 0 and num_chunks >= 2 * RING

    @functools.partial(
        pl.kernel,
        mesh=mesh,
        out_type=jax.ShapeDtypeStruct((NW * num_chunks * CHUNK, d), jnp.float32),
        scratch_types=[
            pltpu.VMEM((num_chunks, CHUNK), jnp.int32),
            pltpu.VMEM((RING, CHUNK, d), jnp.float32),
            pltpu.VMEM((RING, CHUNK, d), jnp.float32),
        ]
        + [pltpu.SemaphoreType.DMA] * (2 * RING),
        compiler_params=pltpu.CompilerParams(use_tc_tiling_on_sc=False),
    )
    def k(idx_hbm, table_hbm, out_hbm, idx_v, gbuf, wbuf, *sems):
        gsem = sems[:RING]
        wsem = sems[RING:]
        wid = lax.axis_index("s") * NC + lax.axis_index("c")
        base = wid * (num_chunks * CHUNK)
        pltpu.sync_copy(idx_hbm.at[wid], idx_v)

        def start_gather(j, b):
            pltpu.async_copy(table_hbm.at[idx_v.at[j]], gbuf.at[b], gsem[b])

        def gather_wait(j, b):
            pltpu.make_async_copy(
                table_hbm.at[idx_v.at[j]], gbuf.at[b], gsem[b]
            ).wait()

        def out_slice(j):
            return out_hbm.at[pl.ds(base + j * CHUNK, CHUNK)]

        def start_wb(j, b):
            pltpu.async_copy(wbuf.at[b], out_slice(j), wsem[b])

        def wb_wait(j, b):
            pltpu.make_async_copy(wbuf.at[b], out_slice(j), wsem[b]).wait()

        def do_scale(b):
            @functools.partial(plsc.parallel_loop, 0, CHUNK, unroll=4)
            def _(i):
                for t in range(d // 16):
                    sl = pl.ds(t * 16, 16)
                    wbuf[b, i, sl] = gbuf[b, i, sl] * scale

        for b in range(RING):
            start_gather(b, b)

        for b in range(RING):
            gather_wait(b, b)
            do_scale(b)
            start_wb(b, b)
            start_gather(RING + b, b)

        def group_body(g, carry):
            for b in range(RING):
                j = g * RING + b
                gather_wait(j, b)
                wb_wait(j - RING, b)
                do_scale(b)
                start_wb(j, b)

                @pl.when(j + RING < num_chunks)
                def _():
                    start_gather(j + RING, b)

            return carry

        lax.fori_loop(1, num_chunks // RING, group_body, 0)

        for b in range(RING):
            wb_wait(num_chunks - RING + b, b)

    return k


def kernel(x, table):
    d = table.shape[1]
    b_total = x.size
    xf = x.reshape(-1).astype(jnp.int32)
    block = NW * CHUNK
    pad = (-b_total) % block
    if pad:
        xf = jnp.concatenate([xf, jnp.zeros((pad,), jnp.int32)])
    num_chunks = xf.size // block
    idx = xf.reshape(NW, num_chunks, CHUNK)
    out = _make_sc_lookup(num_chunks, table.shape[0], d)(idx, table)
    if pad:
        out = out[:b_total]
    return out.reshape(x.shape + (d,))
